# baseline (speedup 1.0000x reference)
import functools

import jax
import jax.numpy as jnp
from jax import lax
from jax.experimental import pallas as pl
from jax.experimental.pallas import tpu as pltpu
from jax.experimental.pallas import tpu_sc as plsc

N_NODES = 10000
N_PAD = 10240
D = 128
N_CORES = 2
N_SUBCORES = 16
N_WORKERS = N_CORES * N_SUBCORES
CHUNK = 128
ROWS_PER_TILE = N_PAD // N_SUBCORES


def _ceil_to(x, m):
    return (x + m - 1) // m * m


def _spmm_sc(x, src, dst, w, zeros, edges_per_tile):
    n_chunks = edges_per_tile // CHUNK
    assert n_chunks % 2 == 0
    mesh = plsc.VectorSubcoreMesh(core_axis_name="c", subcore_axis_name="s")

    @functools.partial(
        pl.kernel,
        out_type=jax.ShapeDtypeStruct((N_CORES, N_PAD, D), jnp.float32),
        mesh=mesh,
        scratch_types=[
            [pltpu.VMEM((CHUNK,), jnp.int32) for _ in range(2)],
            [pltpu.VMEM((CHUNK,), jnp.int32) for _ in range(2)],
            [pltpu.VMEM((CHUNK,), jnp.float32) for _ in range(2)],
            [pltpu.VMEM((CHUNK, D), jnp.float32) for _ in range(2)],
            pltpu.VMEM_SHARED((N_PAD, D), jnp.float32),
            [pltpu.SemaphoreType.DMA for _ in range(2)],
        ],
    )
    def spmm_kernel(x_hbm, src_hbm, dst_hbm, w_hbm, z_hbm, out_hbm,
                    src_v, dst_v, w_v, rows, acc, gsem):
        c = lax.axis_index("c")
        s = lax.axis_index("s")
        wid = c * N_SUBCORES + s

        pltpu.sync_copy(z_hbm.at[pl.ds(s * ROWS_PER_TILE, ROWS_PER_TILE)],
                        acc.at[pl.ds(s * ROWS_PER_TILE, ROWS_PER_TILE)])
        plsc.subcore_barrier()

        tile_base = wid * edges_per_tile

        def stage_and_fire(k, b):
            base = tile_base + k * CHUNK
            pltpu.sync_copy(src_hbm.at[pl.ds(base, CHUNK)], src_v[b])
            pltpu.sync_copy(dst_hbm.at[pl.ds(base, CHUNK)], dst_v[b])
            pltpu.sync_copy(w_hbm.at[pl.ds(base, CHUNK)], w_v[b])
            pltpu.async_copy(x_hbm.at[src_v[b]], rows[b], gsem[b])

        def process(k, b):
            pltpu.make_async_copy(x_hbm.at[src_v[b]], rows[b], gsem[b]).wait()

            def group_body(g, _):
                w16 = w_v[b][pl.ds(g * 16, 16)]
                for e in range(16):
                    wvec = jnp.full((16,), w16[e], jnp.float32)
                    abs_e = g * 16 + e
                    for j in range(D // 16):
                        rows[b][abs_e, pl.ds(j * 16, 16)] = (
                            rows[b][abs_e, pl.ds(j * 16, 16)] * wvec)
                return 0

            lax.fori_loop(0, CHUNK // 16, group_body, 0, unroll=False)
            pltpu.sync_copy(rows[b], acc.at[dst_v[b]], add=True)

        stage_and_fire(0, 0)

        def pair_body(t, _):
            k = t * 2
            stage_and_fire(k + 1, 1)
            process(k, 0)

            @pl.when(k + 2 < n_chunks)
            def _():
                stage_and_fire(k + 2, 0)
            process(k + 1, 1)
            return 0

        lax.fori_loop(0, n_chunks // 2, pair_body, 0, unroll=False)
        plsc.subcore_barrier()
        pltpu.sync_copy(acc.at[pl.ds(s * ROWS_PER_TILE, ROWS_PER_TILE)],
                        out_hbm.at[c].at[pl.ds(s * ROWS_PER_TILE, ROWS_PER_TILE)])

    return spmm_kernel(x, src, dst, w, zeros)


_BLK = 1000


def _mm_body(x_ref, w_ref, o_ref):
    o_ref[...] = jnp.dot(x_ref[...], w_ref[...],
                         preferred_element_type=jnp.float32)


def _mm(x, w):
    n, d_in = x.shape
    d_out = w.shape[1]
    return pl.pallas_call(
        _mm_body,
        grid=(n // _BLK,),
        in_specs=[pl.BlockSpec((_BLK, d_in), lambda i: (i, 0)),
                  pl.BlockSpec((d_in, d_out), lambda i: (0, 0))],
        out_specs=pl.BlockSpec((_BLK, d_out), lambda i: (i, 0)),
        out_shape=jax.ShapeDtypeStruct((n, d_out), jnp.float32),
    )(x, w)


def _relu_mm_body(p0_ref, p1_ref, b_ref, w_ref, o_ref):
    h = jnp.maximum(p0_ref[...] + p1_ref[...] + b_ref[...], 0.0)
    o_ref[...] = jnp.dot(h, w_ref[...], preferred_element_type=jnp.float32)


def _relu_mm(p0, p1, b, w):
    n, d_in = p0.shape
    d_out = w.shape[1]
    return pl.pallas_call(
        _relu_mm_body,
        grid=(n // _BLK,),
        in_specs=[pl.BlockSpec((_BLK, d_in), lambda i: (i, 0)),
                  pl.BlockSpec((_BLK, d_in), lambda i: (i, 0)),
                  pl.BlockSpec((1, d_in), lambda i: (0, 0)),
                  pl.BlockSpec((d_in, d_out), lambda i: (0, 0))],
        out_specs=pl.BlockSpec((_BLK, d_out), lambda i: (i, 0)),
        out_shape=jax.ShapeDtypeStruct((n, d_out), jnp.float32),
    )(p0, p1, b.reshape(1, -1), w)


def _combine_body(q0_ref, q1_ref, b_ref, o_ref):
    o_ref[...] = q0_ref[...] + q1_ref[...] + b_ref[...]


def _combine(q0, q1, b):
    n, d = q0.shape
    return pl.pallas_call(
        _combine_body,
        grid=(n // _BLK,),
        in_specs=[pl.BlockSpec((_BLK, d), lambda i: (i, 0)),
                  pl.BlockSpec((_BLK, d), lambda i: (i, 0)),
                  pl.BlockSpec((1, d), lambda i: (0, 0))],
        out_specs=pl.BlockSpec((_BLK, d), lambda i: (i, 0)),
        out_shape=jax.ShapeDtypeStruct((n, d), jnp.float32),
    )(q0, q1, b.reshape(1, -1))


def kernel(features, edge_index, edge_weight, W1, b1, W2, b2, W3, b3):
    n_edges = edge_index.shape[1]
    e_pad = _ceil_to(n_edges, N_WORKERS * CHUNK * 2)
    edges_per_tile = e_pad // N_WORKERS

    src = jnp.pad(edge_index[0].astype(jnp.int32), (0, e_pad - n_edges))
    dst = jnp.pad(edge_index[1].astype(jnp.int32), (0, e_pad - n_edges))
    w = jnp.pad(edge_weight.astype(jnp.float32), (0, e_pad - n_edges))
    zeros = jnp.zeros((N_PAD, D), jnp.float32)

    xw1 = _mm(features, W1)
    p = _spmm_sc(xw1, src, dst, w, zeros, edges_per_tile)

    W23 = jnp.concatenate([W2, W3], axis=1)
    hw = _relu_mm(p[0, :N_NODES], p[1, :N_NODES], b1, W23)
    q = _spmm_sc(hw, src, dst, w, zeros, edges_per_tile)

    b23 = jnp.concatenate([b2, b3])
    out = _combine(q[0, :N_NODES], q[1, :N_NODES], b23)
    d_out = W2.shape[1]
    return out[:, :d_out], out[:, d_out:]


# speedup vs baseline: 1.1863x; 1.1863x over previous
import functools

import jax
import jax.numpy as jnp
from jax import lax
from jax.experimental import pallas as pl
from jax.experimental.pallas import tpu as pltpu
from jax.experimental.pallas import tpu_sc as plsc

N_NODES = 10000
N_PAD = 10240
D = 128
N_CORES = 2
N_SUBCORES = 16
N_WORKERS = N_CORES * N_SUBCORES
CHUNK = 128
ROWS_PER_TILE = N_PAD // N_SUBCORES


def _ceil_to(x, m):
    return (x + m - 1) // m * m


def _spmm_sc(x, edata, wdata, zeros):
    n_chunks = edata.shape[1]
    assert n_chunks % 2 == 0
    mesh = plsc.VectorSubcoreMesh(core_axis_name="c", subcore_axis_name="s")

    @functools.partial(
        pl.kernel,
        out_type=jax.ShapeDtypeStruct((N_CORES, N_PAD, D), jnp.float32),
        mesh=mesh,
        scratch_types=[
            [pltpu.VMEM((2, CHUNK), jnp.int32) for _ in range(2)],
            [pltpu.VMEM((CHUNK,), jnp.float32) for _ in range(2)],
            [pltpu.VMEM((CHUNK, D), jnp.float32) for _ in range(2)],
            pltpu.VMEM_SHARED((N_PAD, D), jnp.float32),
            [pltpu.SemaphoreType.DMA for _ in range(2)],
            [pltpu.SemaphoreType.DMA for _ in range(2)],
        ],
    )
    def spmm_kernel(x_hbm, e_hbm, w_hbm, z_hbm, out_hbm, ebuf, wbuf, rows, acc, esem, gsem):
        c = lax.axis_index("c")
        s = lax.axis_index("s")
        wid = c * N_SUBCORES + s

        pltpu.sync_copy(z_hbm.at[pl.ds(s * ROWS_PER_TILE, ROWS_PER_TILE)],
                        acc.at[pl.ds(s * ROWS_PER_TILE, ROWS_PER_TILE)])
        plsc.subcore_barrier()

        def fire_ecopy(k, b):
            pltpu.async_copy(e_hbm.at[wid].at[k], ebuf[b], esem[b])
            pltpu.async_copy(w_hbm.at[wid].at[k], wbuf[b], esem[b])

        def wait_ecopy(b):
            pltpu.make_async_copy(e_hbm.at[wid].at[0], ebuf[b], esem[b]).wait()
            pltpu.make_async_copy(w_hbm.at[wid].at[0], wbuf[b], esem[b]).wait()

        def fire_gather(b):
            pltpu.async_copy(x_hbm.at[ebuf[b].at[0]], rows[b], gsem[b])

        def wait_gather(b):
            pltpu.make_async_copy(x_hbm.at[ebuf[b].at[0]], rows[b],
                                  gsem[b]).wait()

        def scale_and_scatter(b):
            def group_body(g, _):
                w16 = wbuf[b][pl.ds(g * 16, 16)]
                for e in range(16):
                    wvec = jnp.full((16,), w16[e], jnp.float32)
                    abs_e = g * 16 + e
                    for j in range(D // 16):
                        rows[b][abs_e, pl.ds(j * 16, 16)] = (
                            rows[b][abs_e, pl.ds(j * 16, 16)] * wvec)
                return 0

            lax.fori_loop(0, CHUNK // 16, group_body, 0, unroll=False)
            pltpu.sync_copy(rows[b], acc.at[ebuf[b].at[1]], add=True)

        fire_ecopy(0, 0)
        wait_ecopy(0)
        fire_gather(0)
        fire_ecopy(1, 1)

        def pair_body(t, _):
            for b in range(2):
                k = t * 2 + b
                bo = 1 - b
                @pl.when(k + 1 < n_chunks)
                def _():
                    wait_ecopy(bo)
                    fire_gather(bo)
                wait_gather(b)
                scale_and_scatter(b)
                @pl.when(k + 2 < n_chunks)
                def _():
                    fire_ecopy(k + 2, b)
            return 0

        lax.fori_loop(0, n_chunks // 2, pair_body, 0, unroll=False)
        plsc.subcore_barrier()
        pltpu.sync_copy(acc.at[pl.ds(s * ROWS_PER_TILE, ROWS_PER_TILE)],
                        out_hbm.at[c].at[pl.ds(s * ROWS_PER_TILE, ROWS_PER_TILE)])

    return spmm_kernel(x, edata, wdata, zeros)


_BLK = 1000


def _mm_body(x_ref, w_ref, o_ref):
    o_ref[...] = jnp.dot(x_ref[...], w_ref[...],
                         preferred_element_type=jnp.float32)


def _mm(x, w):
    n, d_in = x.shape
    d_out = w.shape[1]
    return pl.pallas_call(
        _mm_body,
        grid=(n // _BLK,),
        in_specs=[pl.BlockSpec((_BLK, d_in), lambda i: (i, 0)),
                  pl.BlockSpec((d_in, d_out), lambda i: (0, 0))],
        out_specs=pl.BlockSpec((_BLK, d_out), lambda i: (i, 0)),
        out_shape=jax.ShapeDtypeStruct((n, d_out), jnp.float32),
    )(x, w)


def _relu_mm_body(p0_ref, p1_ref, b_ref, w_ref, o_ref):
    h = jnp.maximum(p0_ref[...] + p1_ref[...] + b_ref[...], 0.0)
    o_ref[...] = jnp.dot(h, w_ref[...], preferred_element_type=jnp.float32)


def _relu_mm(p0, p1, b, w):
    n, d_in = p0.shape
    d_out = w.shape[1]
    return pl.pallas_call(
        _relu_mm_body,
        grid=(n // _BLK,),
        in_specs=[pl.BlockSpec((_BLK, d_in), lambda i: (i, 0)),
                  pl.BlockSpec((_BLK, d_in), lambda i: (i, 0)),
                  pl.BlockSpec((1, d_in), lambda i: (0, 0)),
                  pl.BlockSpec((d_in, d_out), lambda i: (0, 0))],
        out_specs=pl.BlockSpec((_BLK, d_out), lambda i: (i, 0)),
        out_shape=jax.ShapeDtypeStruct((n, d_out), jnp.float32),
    )(p0, p1, b.reshape(1, -1), w)


def _combine_body(q0_ref, q1_ref, b_ref, o_ref):
    o_ref[...] = q0_ref[...] + q1_ref[...] + b_ref[...]


def _combine(q0, q1, b):
    n, d = q0.shape
    return pl.pallas_call(
        _combine_body,
        grid=(n // _BLK,),
        in_specs=[pl.BlockSpec((_BLK, d), lambda i: (i, 0)),
                  pl.BlockSpec((_BLK, d), lambda i: (i, 0)),
                  pl.BlockSpec((1, d), lambda i: (0, 0))],
        out_specs=pl.BlockSpec((_BLK, d), lambda i: (i, 0)),
        out_shape=jax.ShapeDtypeStruct((n, d), jnp.float32),
    )(q0, q1, b.reshape(1, -1))


def kernel(features, edge_index, edge_weight, W1, b1, W2, b2, W3, b3):
    n_edges = edge_index.shape[1]
    e_pad = _ceil_to(n_edges, N_WORKERS * CHUNK * 2)
    n_chunks = e_pad // (N_WORKERS * CHUNK)

    src = jnp.pad(edge_index[0].astype(jnp.int32), (0, e_pad - n_edges))
    dst = jnp.pad(edge_index[1].astype(jnp.int32), (0, e_pad - n_edges))
    w = jnp.pad(edge_weight.astype(jnp.float32), (0, e_pad - n_edges))
    edata = jnp.stack([src, dst])
    edata = edata.reshape(2, N_WORKERS, n_chunks, CHUNK).transpose(1, 2, 0, 3)
    wdata = w.reshape(N_WORKERS, n_chunks, CHUNK)
    zeros = jnp.zeros((N_PAD, D), jnp.float32)

    xw1 = _mm(features, W1)
    p = _spmm_sc(xw1, edata, wdata, zeros)

    W23 = jnp.concatenate([W2, W3], axis=1)
    hw = _relu_mm(p[0, :N_NODES], p[1, :N_NODES], b1, W23)
    q = _spmm_sc(hw, edata, wdata, zeros)

    b23 = jnp.concatenate([b2, b3])
    out = _combine(q[0, :N_NODES], q[1, :N_NODES], b23)
    d_out = W2.shape[1]
    return out[:, :d_out], out[:, d_out:]
